# Initial kernel scaffold; baseline (speedup 1.0000x reference)
import jax
import jax.numpy as jnp
from jax.experimental import pallas as pl


def kernel(features, norm, edge_index, W1, b1, W2, b2):
    raise NotImplementedError("write your pallas kernel here")


# speedup vs baseline: 5.7891x; 5.7891x over previous
import functools

import jax
import jax.numpy as jnp
from jax import lax
from jax.experimental import pallas as pl
from jax.experimental.pallas import tpu as pltpu
from jax.experimental.pallas import tpu_sc as plsc

_N = 10000
_E = 320000
_D = 128
_H = 128
_C = 40

_NW = 32
_CHUNK = 128
_CPW = 80
_E_PAD = _NW * _CPW * _CHUNK
_N_PAD = 10112
_RPT = _N_PAD // 16


def _proj_body(feat_ref, norm_ref, w1_ref, w2_ref, out_ref):
    w12 = jnp.dot(w1_ref[...], w2_ref[...], preferred_element_type=jnp.float32)
    h = feat_ref[...] * norm_ref[...]
    out_ref[...] = jnp.dot(h, w12, preferred_element_type=jnp.float32)


def _mid_body(p_ref, norm_ref, b1_ref, w2_ref, out_ref):
    b12 = jnp.dot(b1_ref[...], w2_ref[...], preferred_element_type=jnp.float32)
    n = norm_ref[...]
    out_ref[...] = (p_ref[0, :_N, :] + p_ref[1, :_N, :]) * (n * n) + n * b12


def _fin_body(p_ref, norm_ref, b2_ref, out_ref):
    out_ref[...] = (p_ref[0, :_N, :] + p_ref[1, :_N, :]) * norm_ref[...] + b2_ref[...]


def _seg_body(table, src_idx, dst_idx, zeros, out, src_v, dst_v, rows_v, acc, sem):
    c = lax.axis_index("c")
    s = lax.axis_index("s")
    wid = s * 2 + c
    r0 = s * _RPT
    pltpu.sync_copy(zeros.at[pl.ds(r0, _RPT)], acc.at[pl.ds(r0, _RPT)])
    pltpu.sync_copy(src_idx.at[wid], src_v)
    pltpu.sync_copy(dst_idx.at[wid], dst_v)
    plsc.subcore_barrier()

    def body(j, carry):
        pltpu.async_copy(table.at[src_v.at[j]], rows_v, sem).wait()
        pltpu.sync_copy(rows_v, acc.at[dst_v.at[j]], add=True)
        return carry

    lax.fori_loop(0, _CPW, body, 0)
    plsc.subcore_barrier()
    pltpu.sync_copy(acc.at[pl.ds(r0, _RPT)], out.at[c, pl.ds(r0, _RPT)])


_seg_call = pl.kernel(
    _seg_body,
    out_type=jax.ShapeDtypeStruct((2, _N_PAD, _C), jnp.float32),
    mesh=plsc.VectorSubcoreMesh(core_axis_name="c", subcore_axis_name="s"),
    scratch_types=[
        pltpu.VMEM((_CPW, _CHUNK), jnp.int32),
        pltpu.VMEM((_CPW, _CHUNK), jnp.int32),
        pltpu.VMEM((_CHUNK, _C), jnp.float32),
        pltpu.VMEM_SHARED((_N_PAD, _C), jnp.float32),
        pltpu.SemaphoreType.DMA,
    ],
    compiler_params=pltpu.CompilerParams(use_tc_tiling_on_sc=False),
)


def kernel(features, norm, edge_index, W1, b1, W2, b2):
    src = edge_index[0]
    dst = edge_index[1]
    src3 = jnp.concatenate(
        [src, jnp.zeros((_E_PAD - _E,), jnp.int32)]).reshape(_NW, _CPW, _CHUNK)
    dst3 = jnp.concatenate(
        [dst, jnp.full((_E_PAD - _E,), _N, jnp.int32)]).reshape(_NW, _CPW, _CHUNK)
    zeros = jnp.zeros((_N_PAD, _C), jnp.float32)
    b1r = b1.reshape(1, _H)
    b2r = b2.reshape(1, _C)

    f = pl.pallas_call(
        _proj_body,
        out_shape=jax.ShapeDtypeStruct((_N, _C), jnp.float32),
    )(features, norm, W1, W2)

    p1 = _seg_call(f, src3, dst3, zeros)

    g = pl.pallas_call(
        _mid_body,
        out_shape=jax.ShapeDtypeStruct((_N, _C), jnp.float32),
    )(p1, norm, b1r, W2)

    p2 = _seg_call(g, src3, dst3, zeros)

    out = pl.pallas_call(
        _fin_body,
        out_shape=jax.ShapeDtypeStruct((_N, _C), jnp.float32),
    )(p2, norm, b2r)

    return out


# speedup vs baseline: 7.0157x; 1.2119x over previous
import functools

import jax
import jax.numpy as jnp
from jax import lax
from jax.experimental import pallas as pl
from jax.experimental.pallas import tpu as pltpu
from jax.experimental.pallas import tpu_sc as plsc

_N = 10000
_E = 320000
_D = 128
_H = 128
_C = 40

_NW = 32
_CHUNK = 128
_CPW = 80
_E_PAD = _NW * _CPW * _CHUNK
_N_PAD = 10112
_RPT = _N_PAD // 16
_NB = 4


def _proj_body(feat_ref, norm_ref, w1_ref, w2_ref, out_ref):
    w12 = jnp.dot(w1_ref[...], w2_ref[...], preferred_element_type=jnp.float32)
    h = feat_ref[...] * norm_ref[...]
    out_ref[...] = jnp.dot(h, w12, preferred_element_type=jnp.float32)


def _mid_body(p_ref, norm_ref, b1_ref, w2_ref, out_ref):
    b12 = jnp.dot(b1_ref[...], w2_ref[...], preferred_element_type=jnp.float32)
    n = norm_ref[...]
    out_ref[...] = (p_ref[0, :_N, :] + p_ref[1, :_N, :]) * (n * n) + n * b12


def _fin_body(p_ref, norm_ref, b2_ref, out_ref):
    out_ref[...] = (p_ref[0, :_N, :] + p_ref[1, :_N, :]) * norm_ref[...] + b2_ref[...]


def _seg_body(table, src_idx, dst_idx, zeros, out, src_v, dst_v, rows_v, acc, sem):
    c = lax.axis_index("c")
    s = lax.axis_index("s")
    wid = s * 2 + c
    r0 = s * _RPT
    pltpu.sync_copy(zeros.at[pl.ds(r0, _RPT)], acc.at[pl.ds(r0, _RPT)])
    pltpu.sync_copy(src_idx.at[wid], src_v)
    pltpu.sync_copy(dst_idx.at[wid], dst_v)
    plsc.subcore_barrier()

    for b in range(_NB):
        pltpu.async_copy(table.at[src_v.at[b]], rows_v.at[b], sem)

    def body(j, carry):
        b = lax.rem(j, _NB)
        pltpu.make_async_copy(table.at[src_v.at[j]], rows_v.at[b], sem).wait()
        pltpu.sync_copy(rows_v.at[b], acc.at[dst_v.at[j]], add=True)

        @pl.when(j + _NB < _CPW)
        def _():
            pltpu.async_copy(table.at[src_v.at[j + _NB]], rows_v.at[b], sem)

        return carry

    lax.fori_loop(0, _CPW, body, 0)
    plsc.subcore_barrier()
    pltpu.sync_copy(acc.at[pl.ds(r0, _RPT)], out.at[c, pl.ds(r0, _RPT)])


_seg_call = pl.kernel(
    _seg_body,
    out_type=jax.ShapeDtypeStruct((2, _N_PAD, _C), jnp.float32),
    mesh=plsc.VectorSubcoreMesh(core_axis_name="c", subcore_axis_name="s"),
    scratch_types=[
        pltpu.VMEM((_CPW, _CHUNK), jnp.int32),
        pltpu.VMEM((_CPW, _CHUNK), jnp.int32),
        pltpu.VMEM((_NB, _CHUNK, _C), jnp.float32),
        pltpu.VMEM_SHARED((_N_PAD, _C), jnp.float32),
        pltpu.SemaphoreType.DMA,
    ],
    compiler_params=pltpu.CompilerParams(use_tc_tiling_on_sc=False),
)


def kernel(features, norm, edge_index, W1, b1, W2, b2):
    src = edge_index[0]
    dst = edge_index[1]
    src3 = jnp.concatenate(
        [src, jnp.zeros((_E_PAD - _E,), jnp.int32)]).reshape(_NW, _CPW, _CHUNK)
    dst3 = jnp.concatenate(
        [dst, jnp.full((_E_PAD - _E,), _N, jnp.int32)]).reshape(_NW, _CPW, _CHUNK)
    zeros = jnp.zeros((_N_PAD, _C), jnp.float32)
    b1r = b1.reshape(1, _H)
    b2r = b2.reshape(1, _C)

    f = pl.pallas_call(
        _proj_body,
        out_shape=jax.ShapeDtypeStruct((_N, _C), jnp.float32),
    )(features, norm, W1, W2)

    p1 = _seg_call(f, src3, dst3, zeros)

    g = pl.pallas_call(
        _mid_body,
        out_shape=jax.ShapeDtypeStruct((_N, _C), jnp.float32),
    )(p1, norm, b1r, W2)

    p2 = _seg_call(g, src3, dst3, zeros)

    out = pl.pallas_call(
        _fin_body,
        out_shape=jax.ShapeDtypeStruct((_N, _C), jnp.float32),
    )(p2, norm, b2r)

    return out
